# baseline (speedup 1.0000x reference)
import functools
import jax
import jax.numpy as jnp
from jax import lax
from jax.experimental import pallas as pl
from jax.experimental.pallas import tpu as pltpu
from jax.experimental.pallas import tpu_sc as plsc

_C = 1000
_D1 = 26
_B = 16384
_NW = 32
_ROWS = _D1 * (_C // 8)
_RW = _ROWS // _NW
_REM = _ROWS - _RW * _NW
_QW = 32768
_NQ = 4


def _sc_onehot(xt_hbm, out_hbm, xrow, buf0, buf1, sem0, sem1):
    wid = lax.axis_index("s") * 2 + lax.axis_index("c")
    start = wid * _RW + jnp.minimum(wid, _REM)
    cnt = _RW + jnp.where(wid < _REM, 1, 0)
    j0 = start // 125
    kt0 = start - j0 * 125

    lanes = lax.iota(jnp.int32, 16)
    ones = jnp.ones((16,), jnp.float32)
    zeros = jnp.zeros((16,), jnp.float32)
    bufs = (buf0, buf1)
    sems = (sem0, sem1)

    pltpu.sync_copy(xt_hbm.at[pl.ds(j0 * _B, _B)], xrow.at[pl.ds(0, _B)])

    def _row(r, carry):
        j, kt, nst = carry
        jsel = nst - 1

        stg = jnp.logical_and(r > 0, kt == 0)

        @pl.when(stg)
        def _():
            pltpu.sync_copy(xt_hbm.at[pl.ds(j * _B, _B)],
                            xrow.at[pl.ds(_B, _B)])

        nst = jnp.where(stg, nst + 1, nst)
        jsel = nst - 1

        row_off = (j * 125 + kt) * (_NQ * _QW)
        for q in range(_NQ):
            c = r * _NQ + q
            b = q % 2
            buf, sem = bufs[b], sems[b]
            dst = out_hbm.at[pl.ds(row_off + q * _QW, _QW)]

            @pl.when(c >= 2)
            def _():
                pltpu.make_async_copy(buf.at[pl.ds(0, _QW)], dst, sem).wait()

            def _z(t, _):
                for u in range(8):
                    buf[pl.ds((t * 8 + u) * 16, 16)] = zeros
                return 0
            lax.fori_loop(0, _QW // 128, _z, 0)

            xb = jsel * _B + q * 4096
            def _s(g, _):
                for u in range(4):
                    gg = g * 4 + u
                    xs = xrow[pl.ds(xb + gg * 16, 16)]
                    m = (xs >> 3) == kt
                    iq = gg * 16 + lanes
                    w = ((iq >> 7) << 10) + ((xs & 7) << 7) + (iq & 127)
                    w = jnp.where(m, w, _QW + lanes)
                    plsc.store_scatter(buf, [w], ones)
                return 0
            lax.fori_loop(0, 4096 // 64, _s, 0)

            pltpu.async_copy(buf.at[pl.ds(0, _QW)], dst, sem)

        kt = kt + 1
        wrap = kt == 125
        j = jnp.where(wrap, j + 1, j)
        kt = jnp.where(wrap, 0, kt)
        return (j, kt, nst)

    j, kt, nst = lax.fori_loop(0, cnt, _row, (j0, kt0, jnp.int32(1)))

    dummy = out_hbm.at[pl.ds(0, _QW)]
    last = cnt * _NQ
    for b in range(2):
        @pl.when(last >= 2 - b)
        def _():
            pltpu.make_async_copy(bufs[b].at[pl.ds(0, _QW)], dummy, sems[b]).wait()


def kernel(x):
    b, c = x.shape
    xt = x.T.astype(jnp.int32).reshape(b * c)
    k = functools.partial(
        pl.kernel,
        mesh=plsc.VectorSubcoreMesh(core_axis_name="c", subcore_axis_name="s"),
        compiler_params=pltpu.CompilerParams(needs_layout_passes=False),
        out_type=jax.ShapeDtypeStruct((b * c * _C,), jnp.float32),
        scratch_types=[
            pltpu.VMEM((2 * _B,), jnp.int32),
            pltpu.VMEM((_QW + 128,), jnp.float32),
            pltpu.VMEM((_QW + 128,), jnp.float32),
            pltpu.SemaphoreType.DMA,
            pltpu.SemaphoreType.DMA,
        ],
    )(_sc_onehot)
    out = k(xt)
    out5 = out.reshape(_D1, _C // 8, _B // 128, 8, 128)
    return out5.transpose(2, 4, 0, 1, 3).reshape(_B, _D1, _C)


# speedup vs baseline: 2.7487x; 2.7487x over previous
import functools
import jax
import jax.numpy as jnp
from jax.experimental import pallas as pl

_C = 1000
_CK = 8


def _onehot_block(x_ref, o_ref):
    kb = pl.program_id(0)
    ids = jax.lax.broadcasted_iota(jnp.int32, o_ref.shape, 1) + kb * _CK
    o_ref[...] = (ids == x_ref[...]).astype(jnp.float32)


def kernel(x):
    b, c = x.shape
    xt = x.T.astype(jnp.int32).reshape(c, 1, b)
    out = pl.pallas_call(
        _onehot_block,
        grid=(_C // _CK,),
        in_specs=[pl.BlockSpec((c, 1, b), lambda kb: (0, 0, 0))],
        out_specs=pl.BlockSpec((c, _CK, b), lambda kb: (0, kb, 0)),
        out_shape=jax.ShapeDtypeStruct((c, _C, b), jnp.float32),
    )(xt)
    return jnp.transpose(out, (2, 0, 1))
